# baseline (speedup 1.0000x reference)
import jax
import jax.numpy as jnp
from jax import lax
from jax.experimental import pallas as pl
from jax.experimental.pallas import tpu as pltpu

_N = 512
_D = 256
_PPROB = 2
_NPAIR = 4
_NP = _PPROB * _NPAIR
_INV_SQRT_D = 1.0 / 16.0
_NEG = -1e30
_POS = 1e30


def _lane(vals):
    return jnp.concatenate(vals, axis=1)


def _to_col(v):
    return jnp.concatenate([v[:, k:k + 1] for k in range(v.shape[1])], axis=0)


def _to_row(v):
    return jnp.concatenate([v[k:k + 1, :] for k in range(v.shape[0])], axis=1)


def _flat16(X):
    return jnp.concatenate([X[i:i + 1, j:j + 1]
                            for i in range(4) for j in range(4)], axis=1)


def _merge_tree(j, W, pe, ue, fm, sub, idx_ref):
    peS, ueS, fmS, MS = [], [], [], []
    for side in range(2):
        L, R = 2 * side, 2 * side + 1
        f0 = jnp.concatenate([fm[L][0], fm[L][1]], axis=0)
        f1 = jnp.concatenate([fm[R][0], fm[R][1]], axis=0)
        f0w = jnp.dot(f0, W, preferred_element_type=jnp.float32)
        sim = lax.dot_general(f0w, f1, (((1,), (1,)), ((), ())),
                              preferred_element_type=jnp.float32) * _INV_SQRT_D
        simflat = jnp.concatenate([sim[0:1, :], sim[1:2, :]], axis=1)
        peL = _lane([pe[L][0], pe[L][0], pe[L][1], pe[L][1]])
        peR = _lane([pe[R][0], pe[R][1], pe[R][0], pe[R][1]])
        ueL = _lane([ue[L][0], ue[L][0], ue[L][1], ue[L][1]])
        ueR = _lane([ue[R][0], ue[R][1], ue[R][0], ue[R][1]])
        peS.append(peL + peR - simflat)
        ueS.append(ueL + ueR)
        f0e = jnp.concatenate([fm[L][0], fm[L][0],
                               fm[L][1], fm[L][1]], axis=0)
        f1t = jnp.concatenate([fm[R][0], fm[R][1],
                               fm[R][0], fm[R][1]], axis=0)
        fmS.append((f0e + f1t) * 0.5)
        rows = []
        for p in range(2):
            for q in range(2):
                rows.append(_lane([sub[L][p][0], sub[L][p][1],
                                   sub[R][q][0], sub[R][q][1]]))
        MS.append(jnp.concatenate(rows, axis=0))

    F0w = jnp.dot(fmS[0], W, preferred_element_type=jnp.float32)
    sim2 = lax.dot_general(F0w, fmS[1], (((1,), (1,)), ((), ())),
                           preferred_element_type=jnp.float32) * _INV_SQRT_D

    total = (_to_col(peS[0]) + peS[1] - sim2
             + 0.1 * (_to_col(ueS[0]) + ueS[1]))

    s16 = _flat16(sim2)
    t16 = _flat16(total)
    sT = _to_col(s16)
    i16 = lax.broadcasted_iota(jnp.int32, (1, 16), 1)
    iT = lax.broadcasted_iota(jnp.int32, (16, 1), 0)
    beats = (sT > s16) | ((sT == s16) & (iT < i16))
    rank = jnp.sum(beats.astype(jnp.int32), axis=0, keepdims=True)
    tmask = jnp.where(rank < 8, t16, _POS)
    tmin = jnp.min(tmask, keepdims=True)
    wi = jnp.min(jnp.where(tmask == tmin, i16, jnp.int32(16)), keepdims=True)
    p_w, q_w = wi // 4, wi % 4

    m0 = lax.broadcasted_iota(jnp.int32, (4, 4), 0) == p_w
    m1 = lax.broadcasted_iota(jnp.int32, (4, 4), 0) == q_w
    left4 = jnp.sum(jnp.where(m0, MS[0], 0), axis=0, keepdims=True)
    right4 = jnp.sum(jnp.where(m1, MS[1], 0), axis=0, keepdims=True)
    idx_ref[j] = jnp.concatenate([left4, right4], axis=1)


def _tourney_kernel(pf_ref, neg_ref, wp_ref, wu_ref, pc_ref, tc_ref,
                    idx_ref, tgt_ref):
    W = wp_ref[...]
    Wu = wu_ref[...]
    riota = lax.broadcasted_iota(jnp.int32, (_N, 1), 0)
    ciota = lax.broadcasted_iota(jnp.int32, (1, _N), 1)
    pairs = [(j, p) for j in range(_PPROB) for p in range(_NPAIR)]

    for j in range(_PPROB):
        tgt_ref[j] = (pc_ref[j] == tc_ref[j, 0, 0]).astype(jnp.float32)

    T8 = []
    for j in range(_PPROB):
        nm = jnp.mean(neg_ref[j], axis=1)
        T8.append(lax.dot_general(nm, Wu, (((1,), (1,)), ((), ())),
                                  preferred_element_type=jnp.float32))

    ms = []
    for k, (j, p) in enumerate(pairs):
        A = pf_ref[j, 2 * p]
        Bm = pf_ref[j, 2 * p + 1]
        AW = jnp.dot(A, W, preferred_element_type=jnp.float32)
        S = lax.dot_general(AW, Bm, (((1,), (1,)), ((), ())),
                            preferred_element_type=jnp.float32) * _INV_SQRT_D
        ms.append(jnp.max(S, axis=1, keepdims=True))

    M = jnp.concatenate(ms, axis=1)
    v1a = jnp.max(M, axis=0, keepdims=True)
    r1a = jnp.min(jnp.where(M == v1a, riota, jnp.int32(_N)),
                  axis=0, keepdims=True)
    A1 = jnp.concatenate(
        [pf_ref[j, 2 * p, pl.ds(r1a[0, k], 1), :]
         for k, (j, p) in enumerate(pairs)], axis=0)
    A1W = jnp.dot(A1, W, preferred_element_type=jnp.float32)
    R1 = jnp.concatenate(
        [lax.dot_general(A1W[k:k + 1, :], pf_ref[j, 2 * p + 1],
                         (((1,), (1,)), ((), ())),
                         preferred_element_type=jnp.float32)
         for k, (j, p) in enumerate(pairs)], axis=0) * _INV_SQRT_D
    v1r = jnp.max(R1, axis=1, keepdims=True)
    c1c = jnp.min(jnp.where(R1 == v1r, ciota, jnp.int32(_N)),
                  axis=1, keepdims=True)
    w2c = jnp.max(jnp.where(ciota == c1c, _NEG, R1),
                  axis=1, keepdims=True)
    MO = jnp.where(riota == r1a, _NEG, M)
    m2a = jnp.max(MO, axis=0, keepdims=True)
    r2oa = jnp.min(jnp.where(MO == m2a, riota, jnp.int32(_N)),
                   axis=0, keepdims=True)
    w2a = _to_row(w2c)
    use_other = (m2a > w2a) | ((m2a == w2a) & (r2oa < r1a))
    r2a = jnp.where(use_other, r2oa, r1a)
    A2 = jnp.concatenate(
        [pf_ref[j, 2 * p, pl.ds(r2a[0, k], 1), :]
         for k, (j, p) in enumerate(pairs)], axis=0)
    A2W = jnp.dot(A2, W, preferred_element_type=jnp.float32)
    R2 = jnp.concatenate(
        [lax.dot_general(A2W[k:k + 1, :], pf_ref[j, 2 * p + 1],
                         (((1,), (1,)), ((), ())),
                         preferred_element_type=jnp.float32)
         for k, (j, p) in enumerate(pairs)], axis=0) * _INV_SQRT_D
    same_rc = _to_col(jnp.where(r2a == r1a, 1, 0))
    row2m = jnp.where((ciota == c1c) & (same_rc == 1), _NEG, R2)
    v2r = jnp.max(row2m, axis=1, keepdims=True)
    c2c = jnp.min(jnp.where(row2m == v2r, ciota, jnp.int32(_N)),
                  axis=1, keepdims=True)
    v2a = _to_row(v2r)

    B1 = jnp.concatenate(
        [pf_ref[j, 2 * p + 1, pl.ds(c1c[k, 0], 1), :]
         for k, (j, p) in enumerate(pairs)], axis=0)
    B2 = jnp.concatenate(
        [pf_ref[j, 2 * p + 1, pl.ds(c2c[k, 0], 1), :]
         for k, (j, p) in enumerate(pairs)], axis=0)
    T0 = jnp.concatenate(
        [T8[j][2 * p:2 * p + 1, :] for (j, p) in pairs], axis=0)
    T1 = jnp.concatenate(
        [T8[j][2 * p + 1:2 * p + 2, :] for (j, p) in pairs], axis=0)
    ue1c = (jnp.sum(A1 * T0, axis=1, keepdims=True)
            + jnp.sum(B1 * T1, axis=1, keepdims=True)) * _INV_SQRT_D
    ue2c = (jnp.sum(A2 * T0, axis=1, keepdims=True)
            + jnp.sum(B2 * T1, axis=1, keepdims=True)) * _INV_SQRT_D
    FM1 = (A1 + B1) * 0.5
    FM2 = (A2 + B2) * 0.5

    for j in range(_PPROB):
        pe, ue, fm, sub = [], [], [], []
        for p in range(_NPAIR):
            k = j * _NPAIR + p
            pe.append((-v1a[0:1, k:k + 1], -v2a[0:1, k:k + 1]))
            ue.append((ue1c[k:k + 1, 0:1], ue2c[k:k + 1, 0:1]))
            fm.append((FM1[k:k + 1, :], FM2[k:k + 1, :]))
            sub.append(((r1a[0:1, k:k + 1], c1c[k:k + 1, 0:1]),
                        (r2a[0:1, k:k + 1], c2c[k:k + 1, 0:1])))
        _merge_tree(j, W, pe, ue, fm, sub, idx_ref)


def kernel(pos_fea, neg_fea, pos_classes, neg_classes, target_class,
           training, W_pair, W_unary):
    B, KBAG, N, D = pos_fea.shape
    neg4 = neg_fea.reshape(B, KBAG, neg_fea.shape[1], D)
    tc3 = target_class.astype(jnp.int32).reshape(B, 1, 1)
    pc3 = pos_classes.astype(jnp.int32)
    P = _PPROB

    idx, tgt = pl.pallas_call(
        _tourney_kernel,
        grid=(B // P,),
        in_specs=[
            pl.BlockSpec((P, KBAG, N, D), lambda g: (g, 0, 0, 0)),
            pl.BlockSpec((P, KBAG, neg4.shape[2], D), lambda g: (g, 0, 0, 0)),
            pl.BlockSpec((D, D), lambda g: (0, 0)),
            pl.BlockSpec((D, D), lambda g: (0, 0)),
            pl.BlockSpec((P, KBAG, N), lambda g: (g, 0, 0)),
            pl.BlockSpec((P, 1, 1), lambda g: (g, 0, 0)),
        ],
        out_specs=[
            pl.BlockSpec((P, 1, KBAG), lambda g: (g, 0, 0)),
            pl.BlockSpec((P, KBAG, N), lambda g: (g, 0, 0)),
        ],
        out_shape=[
            jax.ShapeDtypeStruct((B, 1, KBAG), jnp.int32),
            jax.ShapeDtypeStruct((B, KBAG, N), jnp.float32),
        ],
        compiler_params=pltpu.CompilerParams(
            dimension_semantics=("parallel",)),
    )(pos_fea, neg4, W_pair, W_unary, pc3, tc3)

    return idx.reshape(B, KBAG), tgt


# speedup vs baseline: 1.2577x; 1.2577x over previous
import jax
import jax.numpy as jnp
from jax import lax
from jax.experimental import pallas as pl
from jax.experimental.pallas import tpu as pltpu

_N = 512
_D = 256
_PPROB = 2
_NPAIR = 4
_NP = _PPROB * _NPAIR
_INV_SQRT_D = 1.0 / 16.0
_NEG = -1e30
_POS = 1e30


def _lane(vals):
    return jnp.concatenate(vals, axis=1)


def _to_col(v):
    return jnp.concatenate([v[:, k:k + 1] for k in range(v.shape[1])], axis=0)


def _to_row(v):
    return jnp.concatenate([v[k:k + 1, :] for k in range(v.shape[0])], axis=1)


def _flat16(X):
    return jnp.concatenate([X[i:i + 1, j:j + 1]
                            for i in range(4) for j in range(4)], axis=1)


def _merge_tree(j, W, pe, ue, fm, sub, idx_ref):
    peS, ueS, fmS, MS = [], [], [], []
    for side in range(2):
        L, R = 2 * side, 2 * side + 1
        f0 = jnp.concatenate([fm[L][0], fm[L][1]], axis=0)
        f1 = jnp.concatenate([fm[R][0], fm[R][1]], axis=0)
        f0w = jnp.dot(f0, W, preferred_element_type=jnp.float32)
        sim = lax.dot_general(f0w, f1, (((1,), (1,)), ((), ())),
                              preferred_element_type=jnp.float32) * _INV_SQRT_D
        simflat = jnp.concatenate([sim[0:1, :], sim[1:2, :]], axis=1)
        peL = _lane([pe[L][0], pe[L][0], pe[L][1], pe[L][1]])
        peR = _lane([pe[R][0], pe[R][1], pe[R][0], pe[R][1]])
        ueL = _lane([ue[L][0], ue[L][0], ue[L][1], ue[L][1]])
        ueR = _lane([ue[R][0], ue[R][1], ue[R][0], ue[R][1]])
        peS.append(peL + peR - simflat)
        ueS.append(ueL + ueR)
        f0e = jnp.concatenate([fm[L][0], fm[L][0],
                               fm[L][1], fm[L][1]], axis=0)
        f1t = jnp.concatenate([fm[R][0], fm[R][1],
                               fm[R][0], fm[R][1]], axis=0)
        fmS.append((f0e + f1t) * 0.5)
        rows = []
        for p in range(2):
            for q in range(2):
                rows.append(_lane([sub[L][p][0], sub[L][p][1],
                                   sub[R][q][0], sub[R][q][1]]))
        MS.append(jnp.concatenate(rows, axis=0))

    F0w = jnp.dot(fmS[0], W, preferred_element_type=jnp.float32)
    sim2 = lax.dot_general(F0w, fmS[1], (((1,), (1,)), ((), ())),
                           preferred_element_type=jnp.float32) * _INV_SQRT_D

    total = (_to_col(peS[0]) + peS[1] - sim2
             + 0.1 * (_to_col(ueS[0]) + ueS[1]))

    s16 = _flat16(sim2)
    t16 = _flat16(total)
    sT = _to_col(s16)
    i16 = lax.broadcasted_iota(jnp.int32, (1, 16), 1)
    iT = lax.broadcasted_iota(jnp.int32, (16, 1), 0)
    beats = (sT > s16) | ((sT == s16) & (iT < i16))
    rank = jnp.sum(beats.astype(jnp.int32), axis=0, keepdims=True)
    tmask = jnp.where(rank < 8, t16, _POS)
    tmin = jnp.min(tmask, keepdims=True)
    wi = jnp.min(jnp.where(tmask == tmin, i16, jnp.int32(16)), keepdims=True)
    p_w, q_w = wi // 4, wi % 4

    m0 = lax.broadcasted_iota(jnp.int32, (4, 4), 0) == p_w
    m1 = lax.broadcasted_iota(jnp.int32, (4, 4), 0) == q_w
    left4 = jnp.sum(jnp.where(m0, MS[0], 0), axis=0, keepdims=True)
    right4 = jnp.sum(jnp.where(m1, MS[1], 0), axis=0, keepdims=True)
    idx_ref[j] = jnp.concatenate([left4, right4], axis=1)


def _tourney_kernel(pf_ref, neg_ref, wp_ref, wu_ref, pc_ref, tc_ref,
                    idx_ref, tgt_ref, *s_refs):
    W = wp_ref[...]
    Wu = wu_ref[...]
    riota = lax.broadcasted_iota(jnp.int32, (_N, 1), 0)
    ciota = lax.broadcasted_iota(jnp.int32, (1, _N), 1)
    pairs = [(j, p) for j in range(_PPROB) for p in range(_NPAIR)]

    for j in range(_PPROB):
        tgt_ref[j] = (pc_ref[j] == tc_ref[j, 0, 0]).astype(jnp.float32)

    T8 = []
    for j in range(_PPROB):
        nm = jnp.mean(neg_ref[j], axis=1)
        T8.append(lax.dot_general(nm, Wu, (((1,), (1,)), ((), ())),
                                  preferred_element_type=jnp.float32))

    ms = []
    for k, (j, p) in enumerate(pairs):
        A = pf_ref[j, 2 * p]
        Bm = pf_ref[j, 2 * p + 1]
        AW = jnp.dot(A, W, preferred_element_type=jnp.float32)
        S = lax.dot_general(AW, Bm, (((1,), (1,)), ((), ())),
                            preferred_element_type=jnp.float32) * _INV_SQRT_D
        s_refs[k][...] = S
        ms.append(jnp.max(S, axis=1, keepdims=True))

    M = jnp.concatenate(ms, axis=1)
    v1a = jnp.max(M, axis=0, keepdims=True)
    r1a = jnp.min(jnp.where(M == v1a, riota, jnp.int32(_N)),
                  axis=0, keepdims=True)
    R1 = jnp.concatenate(
        [s_refs[k][pl.ds(r1a[0, k], 1), :] for k in range(_NP)],
        axis=0)
    v1c = _to_col(v1a)
    c1c = jnp.min(jnp.where(R1 == v1c, ciota, jnp.int32(_N)),
                  axis=1, keepdims=True)
    w2c = jnp.max(jnp.where(ciota == c1c, _NEG, R1),
                  axis=1, keepdims=True)
    MO = jnp.where(riota == r1a, _NEG, M)
    m2a = jnp.max(MO, axis=0, keepdims=True)
    r2oa = jnp.min(jnp.where(MO == m2a, riota, jnp.int32(_N)),
                   axis=0, keepdims=True)
    w2a = _to_row(w2c)
    use_other = (m2a > w2a) | ((m2a == w2a) & (r2oa < r1a))
    v2a = jnp.where(use_other, m2a, w2a)
    r2a = jnp.where(use_other, r2oa, r1a)
    R2 = jnp.concatenate(
        [s_refs[k][pl.ds(r2a[0, k], 1), :] for k in range(_NP)],
        axis=0)
    same_rc = _to_col(jnp.where(r2a == r1a, 1, 0))
    row2m = jnp.where((ciota == c1c) & (same_rc == 1), _NEG, R2)
    v2cc = _to_col(v2a)
    c2c = jnp.min(jnp.where(row2m == v2cc, ciota, jnp.int32(_N)),
                  axis=1, keepdims=True)

    A1 = jnp.concatenate(
        [pf_ref[j, 2 * p, pl.ds(r1a[0, k], 1), :]
         for k, (j, p) in enumerate(pairs)], axis=0)
    B1 = jnp.concatenate(
        [pf_ref[j, 2 * p + 1, pl.ds(c1c[k, 0], 1), :]
         for k, (j, p) in enumerate(pairs)], axis=0)
    A2 = jnp.concatenate(
        [pf_ref[j, 2 * p, pl.ds(r2a[0, k], 1), :]
         for k, (j, p) in enumerate(pairs)], axis=0)
    B2 = jnp.concatenate(
        [pf_ref[j, 2 * p + 1, pl.ds(c2c[k, 0], 1), :]
         for k, (j, p) in enumerate(pairs)], axis=0)
    T0 = jnp.concatenate(
        [T8[j][2 * p:2 * p + 1, :] for (j, p) in pairs], axis=0)
    T1 = jnp.concatenate(
        [T8[j][2 * p + 1:2 * p + 2, :] for (j, p) in pairs], axis=0)
    ue1c = (jnp.sum(A1 * T0, axis=1, keepdims=True)
            + jnp.sum(B1 * T1, axis=1, keepdims=True)) * _INV_SQRT_D
    ue2c = (jnp.sum(A2 * T0, axis=1, keepdims=True)
            + jnp.sum(B2 * T1, axis=1, keepdims=True)) * _INV_SQRT_D
    FM1 = (A1 + B1) * 0.5
    FM2 = (A2 + B2) * 0.5

    for j in range(_PPROB):
        pe, ue, fm, sub = [], [], [], []
        for p in range(_NPAIR):
            k = j * _NPAIR + p
            pe.append((-v1a[0:1, k:k + 1], -v2a[0:1, k:k + 1]))
            ue.append((ue1c[k:k + 1, 0:1], ue2c[k:k + 1, 0:1]))
            fm.append((FM1[k:k + 1, :], FM2[k:k + 1, :]))
            sub.append(((r1a[0:1, k:k + 1], c1c[k:k + 1, 0:1]),
                        (r2a[0:1, k:k + 1], c2c[k:k + 1, 0:1])))
        _merge_tree(j, W, pe, ue, fm, sub, idx_ref)


def kernel(pos_fea, neg_fea, pos_classes, neg_classes, target_class,
           training, W_pair, W_unary):
    B, KBAG, N, D = pos_fea.shape
    neg4 = neg_fea.reshape(B, KBAG, neg_fea.shape[1], D)
    tc3 = target_class.astype(jnp.int32).reshape(B, 1, 1)
    pc3 = pos_classes.astype(jnp.int32)
    P = _PPROB

    idx, tgt = pl.pallas_call(
        _tourney_kernel,
        grid=(B // P,),
        in_specs=[
            pl.BlockSpec((P, KBAG, N, D), lambda g: (g, 0, 0, 0)),
            pl.BlockSpec((P, KBAG, neg4.shape[2], D), lambda g: (g, 0, 0, 0)),
            pl.BlockSpec((D, D), lambda g: (0, 0)),
            pl.BlockSpec((D, D), lambda g: (0, 0)),
            pl.BlockSpec((P, KBAG, N), lambda g: (g, 0, 0)),
            pl.BlockSpec((P, 1, 1), lambda g: (g, 0, 0)),
        ],
        out_specs=[
            pl.BlockSpec((P, 1, KBAG), lambda g: (g, 0, 0)),
            pl.BlockSpec((P, KBAG, N), lambda g: (g, 0, 0)),
        ],
        out_shape=[
            jax.ShapeDtypeStruct((B, 1, KBAG), jnp.int32),
            jax.ShapeDtypeStruct((B, KBAG, N), jnp.float32),
        ],
        scratch_shapes=[pltpu.VMEM((N, N), jnp.float32)
                        for _ in range(P * _NPAIR)],
        compiler_params=pltpu.CompilerParams(
            dimension_semantics=("parallel",)),
    )(pos_fea, neg4, W_pair, W_unary, pc3, tc3)

    return idx.reshape(B, KBAG), tgt
